# baseline (speedup 1.0000x reference)
import jax
import jax.numpy as jnp
from jax.experimental import pallas as pl

_B, _P, _C = 16384, 1000, 4
_BB = 512


def _onehot_body(xt_ref, o_ref):
    xt = xt_ref[...]
    for v in (1, 2, 3, 4):
        o_ref[:, v - 1, :] = (xt == jnp.float32(v)).astype(jnp.float32)


def kernel(x):
    xt = x.T
    out_t = pl.pallas_call(
        _onehot_body,
        grid=(_B // _BB,),
        in_specs=[pl.BlockSpec((_P, _BB), lambda i: (0, i))],
        out_specs=pl.BlockSpec((_P, _C, _BB), lambda i: (0, 0, i)),
        out_shape=jax.ShapeDtypeStruct((_P, _C, _B), jnp.float32),
    )(xt)
    return out_t.transpose(2, 0, 1)


# speedup vs baseline: 1.0207x; 1.0207x over previous
import jax
import jax.numpy as jnp
from jax.experimental import pallas as pl

_B, _P, _C = 16384, 1000, 4
_PP = 40


def _onehot_body(xt_ref, o_ref):
    xt = xt_ref[...]
    for v in (1, 2, 3, 4):
        o_ref[:, v - 1, :] = (xt == jnp.float32(v)).astype(jnp.float32)


def kernel(x):
    xt = x.T
    out_t = pl.pallas_call(
        _onehot_body,
        grid=(_P // _PP,),
        in_specs=[pl.BlockSpec((_PP, _B), lambda i: (i, 0))],
        out_specs=pl.BlockSpec((_PP, _C, _B), lambda i: (i, 0, 0)),
        out_shape=jax.ShapeDtypeStruct((_P, _C, _B), jnp.float32),
    )(xt)
    return out_t.transpose(2, 0, 1)


# speedup vs baseline: 1.0398x; 1.0187x over previous
import jax
import jax.numpy as jnp
from jax.experimental import pallas as pl

_B, _P, _C = 16384, 1000, 4
_BB = 1024


def _onehot_body(xt_ref, o_ref):
    xt = xt_ref[...]
    for v in (1, 2, 3, 4):
        o_ref[:, v - 1, :] = (xt == jnp.float32(v)).astype(jnp.float32)


def kernel(x):
    xt = x.T
    out_t = pl.pallas_call(
        _onehot_body,
        grid=(_B // _BB,),
        in_specs=[pl.BlockSpec((_P, _BB), lambda i: (0, i))],
        out_specs=pl.BlockSpec((_P, _C, _BB), lambda i: (0, 0, i)),
        out_shape=jax.ShapeDtypeStruct((_P, _C, _B), jnp.float32),
    )(xt)
    return out_t.transpose(2, 0, 1)
